# Initial kernel scaffold; baseline (speedup 1.0000x reference)
import jax
import jax.numpy as jnp
from jax.experimental import pallas as pl


def kernel(token_tag, tok_table, tag_table):
    raise NotImplementedError("write your pallas kernel here")


# speedup vs baseline: 1.4767x; 1.4767x over previous
import functools

import jax
import jax.numpy as jnp
from jax import lax
from jax.experimental import pallas as pl
from jax.experimental.pallas import tpu as pltpu
from jax.experimental.pallas import tpu_sc as plsc

_NC = 2
_NS = 16
_NW = _NC * _NS

_SUB = 128
_CHUNK = 1024


def _gather_call(idx_flat, table):
    n, = idx_flat.shape
    _, d = table.shape
    n_per_w = n // _NW
    n_chunks = n_per_w // _CHUNK
    n_sub = _CHUNK // _SUB

    mesh = plsc.VectorSubcoreMesh(
        core_axis_name="c", subcore_axis_name="s",
        num_cores=_NC, num_subcores=_NS)

    @functools.partial(
        pl.kernel,
        out_type=jax.ShapeDtypeStruct((n, d), jnp.float32),
        mesh=mesh,
        scratch_types=[
            pltpu.VMEM((n_per_w,), jnp.int32),
            pltpu.VMEM((_CHUNK, d), jnp.float32),
            pltpu.SemaphoreType.DMA,
        ],
        compiler_params=pltpu.CompilerParams(use_tc_tiling_on_sc=False),
    )
    def k(idx_hbm, table_hbm, out_hbm, idx_v, rows_v, gsem):
        wid = lax.axis_index("s") * _NC + lax.axis_index("c")
        base = pl.multiple_of(wid * n_per_w, 8)
        pltpu.sync_copy(idx_hbm.at[pl.ds(base, n_per_w)], idx_v)

        def chunk(ci, carry):
            off = pl.multiple_of(ci * _CHUNK, 8)
            copies = []
            for t in range(n_sub):
                copies.append(pltpu.async_copy(
                    table_hbm.at[idx_v.at[pl.ds(off + t * _SUB, _SUB)]],
                    rows_v.at[pl.ds(t * _SUB, _SUB)],
                    gsem))
            for cp in copies:
                cp.wait()
            pltpu.sync_copy(rows_v, out_hbm.at[pl.ds(base + off, _CHUNK)])
            return carry

        lax.fori_loop(0, n_chunks, chunk, 0)

    return k(idx_flat, table)


def kernel(token_tag, tok_table, tag_table):
    b, s = token_tag.shape
    _, d = tok_table.shape
    idx = token_tag.reshape(b * s)
    out = _gather_call(idx, tok_table)
    return out.reshape(b, s, d)
